# baseline (speedup 1.0000x reference)
import jax
import jax.numpy as jnp
from jax import lax
from jax.experimental import pallas as pl
from jax.experimental.pallas import tpu as pltpu
from jax.experimental.pallas import tpu_sc as plsc

N_NODES = 10000
N_EDGES = 320000
D_FEAT = 128

NC = 2
NS = 16
NW = NC * NS

CHUNK = 128
ROWS_PER_W = 80
N_ROWS = NW * ROWS_PER_W
E_PAD = N_ROWS * CHUNK
ZROWS = 632
N_ACC = NS * ZROWS


def _sc_accumulate(x_hbm, src_hbm, dst_hbm, part_hbm,
                   acc_sh, src_v, dst_v, rows_v, gsem):
    c = lax.axis_index("c")
    s = lax.axis_index("s")
    wid = s * NC + c

    def _zero_vmem(i, _):
        for j in range(8):
            rows_v[i, pl.ds(j * 16, 16)] = jnp.zeros((16,), jnp.float32)
        return 0
    lax.fori_loop(0, CHUNK, _zero_vmem, 0)
    zbase = s * ZROWS
    for k in range(4):
        pltpu.sync_copy(rows_v, acc_sh.at[pl.ds(zbase + k * CHUNK, CHUNK), :])
    pltpu.sync_copy(rows_v.at[pl.ds(0, ZROWS - 4 * CHUNK), :],
                    acc_sh.at[pl.ds(zbase + 4 * CHUNK, ZROWS - 4 * CHUNK), :])
    plsc.subcore_barrier()

    rbase = wid * ROWS_PER_W

    def _edge_step(i, _):
        row = rbase + i
        pltpu.sync_copy(src_hbm.at[row, :], src_v)
        pltpu.sync_copy(dst_hbm.at[row, :], dst_v)
        pltpu.async_copy(x_hbm.at[src_v], rows_v, gsem).wait()
        pltpu.sync_copy(rows_v, acc_sh.at[dst_v], add=True)
        return 0
    lax.fori_loop(0, ROWS_PER_W, _edge_step, 0)
    plsc.subcore_barrier()

    wbase = s * ZROWS
    pltpu.sync_copy(acc_sh.at[pl.ds(wbase, ZROWS), :],
                    part_hbm.at[c, pl.ds(wbase, ZROWS), :])


def _combine_body(p_ref, o_ref):
    o_ref[...] = p_ref[0] + p_ref[1]


@jax.jit
def kernel(x, edge_index):
    n_pad = E_PAD - N_EDGES
    pad_dst = N_NODES + (jnp.arange(n_pad, dtype=jnp.int32)
                         % (N_ACC - N_NODES))
    src2d = jnp.concatenate(
        [edge_index[0], jnp.zeros((n_pad,), jnp.int32)]).reshape(N_ROWS, CHUNK)
    dst2d = jnp.concatenate(
        [edge_index[1], pad_dst]).reshape(N_ROWS, CHUNK)

    mesh = plsc.VectorSubcoreMesh(core_axis_name="c", subcore_axis_name="s",
                                  num_cores=NC, num_subcores=NS)
    partials = pl.kernel(
        _sc_accumulate,
        out_type=jax.ShapeDtypeStruct((NC, N_ACC, D_FEAT), jnp.float32),
        mesh=mesh,
        scratch_types=[
            pltpu.VMEM_SHARED((N_ACC, D_FEAT), jnp.float32),
            pltpu.VMEM((CHUNK,), jnp.int32),
            pltpu.VMEM((CHUNK,), jnp.int32),
            pltpu.VMEM((CHUNK, D_FEAT), jnp.float32),
            pltpu.SemaphoreType.DMA,
        ],
    )(x, src2d, dst2d)

    out = pl.pallas_call(
        _combine_body,
        out_shape=jax.ShapeDtypeStruct((N_NODES, D_FEAT), jnp.float32),
        grid=(10,),
        in_specs=[pl.BlockSpec((NC, N_NODES // 10, D_FEAT),
                               lambda i: (0, i, 0))],
        out_specs=pl.BlockSpec((N_NODES // 10, D_FEAT), lambda i: (i, 0)),
    )(partials)
    return out


# speedup vs baseline: 3.6885x; 3.6885x over previous
import jax
import jax.numpy as jnp
from jax import lax
from jax.experimental import pallas as pl
from jax.experimental.pallas import tpu as pltpu
from jax.experimental.pallas import tpu_sc as plsc

N_NODES = 10000
N_EDGES = 320000
D_FEAT = 128

NC = 2
NS = 16
NW = NC * NS

CHUNK = 128
ROWS_PER_W = 80
N_ROWS = NW * ROWS_PER_W
E_PAD = N_ROWS * CHUNK
ZROWS = 640
N_ACC = NS * ZROWS


def _sc_accumulate(x_hbm, src_hbm, dst_hbm, part_hbm,
                   acc_sh, src_v, dst_v, rows_v, gsem):
    c = lax.axis_index("c")
    s = lax.axis_index("s")
    wid = s * NC + c

    def _zero_vmem(i, _):
        for j in range(8):
            rows_v[0, i, pl.ds(j * 16, 16)] = jnp.zeros((16,), jnp.float32)
        return 0
    lax.fori_loop(0, CHUNK, _zero_vmem, 0)
    zbase = s * ZROWS
    for k in range(ZROWS // CHUNK):
        pltpu.sync_copy(rows_v.at[0],
                        acc_sh.at[pl.ds(zbase + k * CHUNK, CHUNK), :])
    plsc.subcore_barrier()

    rbase = wid * ROWS_PER_W
    FIFTH = 16

    def _gather_start(b, i):
        pltpu.async_copy(x_hbm.at[src_v.at[i]], rows_v.at[b], gsem)

    def _gather_wait(b, i):
        pltpu.make_async_copy(x_hbm.at[src_v.at[i]], rows_v.at[b],
                              gsem).wait()

    for f in range(ROWS_PER_W // FIFTH):
        fb = rbase + f * FIFTH
        pltpu.sync_copy(src_hbm.at[pl.ds(fb, FIFTH), :], src_v)
        pltpu.sync_copy(dst_hbm.at[pl.ds(fb, FIFTH), :], dst_v)
        _gather_start(0, 0)

        def _pair(pr, _):
            i0 = 2 * pr
            _gather_wait(0, i0)
            _gather_start(1, i0 + 1)
            pltpu.sync_copy(rows_v.at[0], acc_sh.at[dst_v.at[i0]], add=True)
            _gather_wait(1, i0 + 1)

            @pl.when(pr < FIFTH // 2 - 1)
            def _next_even():
                _gather_start(0, i0 + 2)
            pltpu.sync_copy(rows_v.at[1], acc_sh.at[dst_v.at[i0 + 1]],
                            add=True)
            return 0

        lax.fori_loop(0, FIFTH // 2, _pair, 0)
    plsc.subcore_barrier()

    wbase = s * ZROWS
    pltpu.sync_copy(acc_sh.at[pl.ds(wbase, ZROWS), :],
                    part_hbm.at[c, pl.ds(wbase, ZROWS), :])


def _combine_body(p_ref, o_ref):
    o_ref[...] = p_ref[0] + p_ref[1]


@jax.jit
def kernel(x, edge_index):
    n_pad = E_PAD - N_EDGES
    ar = jnp.arange(n_pad, dtype=jnp.int32)
    pad_src = ar % N_NODES
    pad_dst = N_NODES + (ar % (N_ACC - N_NODES))
    src2d = jnp.concatenate(
        [edge_index[0], pad_src]).reshape(N_ROWS, CHUNK)
    dst2d = jnp.concatenate(
        [edge_index[1], pad_dst]).reshape(N_ROWS, CHUNK)

    mesh = plsc.VectorSubcoreMesh(core_axis_name="c", subcore_axis_name="s",
                                  num_cores=NC, num_subcores=NS)
    partials = pl.kernel(
        _sc_accumulate,
        out_type=jax.ShapeDtypeStruct((NC, N_ACC, D_FEAT), jnp.float32),
        mesh=mesh,
        scratch_types=[
            pltpu.VMEM_SHARED((N_ACC, D_FEAT), jnp.float32),
            pltpu.VMEM((16, CHUNK), jnp.int32),
            pltpu.VMEM((16, CHUNK), jnp.int32),
            pltpu.VMEM((2, CHUNK, D_FEAT), jnp.float32),
            pltpu.SemaphoreType.DMA,
        ],
    )(x, src2d, dst2d)

    out = pl.pallas_call(
        _combine_body,
        out_shape=jax.ShapeDtypeStruct((N_NODES, D_FEAT), jnp.float32),
        grid=(10,),
        in_specs=[pl.BlockSpec((NC, N_NODES // 10, D_FEAT),
                               lambda i: (0, i, 0))],
        out_specs=pl.BlockSpec((N_NODES // 10, D_FEAT), lambda i: (i, 0)),
    )(partials)
    return out


# speedup vs baseline: 3.6982x; 1.0026x over previous
import jax
import jax.numpy as jnp
from jax import lax
from jax.experimental import pallas as pl
from jax.experimental.pallas import tpu as pltpu
from jax.experimental.pallas import tpu_sc as plsc

N_NODES = 10000
N_EDGES = 320000
D_FEAT = 128

NC = 2
NS = 16
NW = NC * NS

CHUNK = 128
ROWS_PER_W = 80
N_ROWS = NW * ROWS_PER_W
E_PAD = N_ROWS * CHUNK
ZROWS = 640
N_ACC = NS * ZROWS


def _sc_accumulate(x_hbm, src_hbm, dst_hbm, part_hbm,
                   acc_sh, src_v, dst_v, rows_v, gsem, ssem):
    c = lax.axis_index("c")
    s = lax.axis_index("s")
    wid = s * NC + c

    def _zero_vmem(i, _):
        for j in range(8):
            rows_v[0, i, pl.ds(j * 16, 16)] = jnp.zeros((16,), jnp.float32)
        return 0
    lax.fori_loop(0, CHUNK, _zero_vmem, 0)
    zbase = s * ZROWS
    for k in range(ZROWS // CHUNK):
        pltpu.sync_copy(rows_v.at[0],
                        acc_sh.at[pl.ds(zbase + k * CHUNK, CHUNK), :])
    plsc.subcore_barrier()

    rbase = wid * ROWS_PER_W
    FIFTH = 16

    def _gather_start(b, i):
        pltpu.async_copy(x_hbm.at[src_v.at[i]], rows_v.at[b], gsem.at[b])

    def _gather_wait(b, i):
        pltpu.make_async_copy(x_hbm.at[src_v.at[i]], rows_v.at[b],
                              gsem.at[b]).wait()

    def _scatter_start(b, i):
        pltpu.async_copy(rows_v.at[b], acc_sh.at[dst_v.at[i]], ssem.at[b],
                         add=True)

    def _scatter_wait(b, i):
        pltpu.make_async_copy(rows_v.at[b], acc_sh.at[dst_v.at[i]],
                              ssem.at[b]).wait()

    for f in range(ROWS_PER_W // FIFTH):
        fb = rbase + f * FIFTH
        pltpu.sync_copy(src_hbm.at[pl.ds(fb, FIFTH), :], src_v)
        pltpu.sync_copy(dst_hbm.at[pl.ds(fb, FIFTH), :], dst_v)
        _gather_start(0, 0)

        def _pair(pr, _):
            i0 = 2 * pr
            _gather_wait(0, i0)
            _scatter_start(0, i0)

            @pl.when(pr > 0)
            def _free_buf1():
                _scatter_wait(1, i0 - 1)
            _gather_start(1, i0 + 1)
            _gather_wait(1, i0 + 1)
            _scatter_start(1, i0 + 1)
            _scatter_wait(0, i0)

            @pl.when(pr < FIFTH // 2 - 1)
            def _next_even():
                _gather_start(0, i0 + 2)
            return 0

        lax.fori_loop(0, FIFTH // 2, _pair, 0)
        _scatter_wait(1, FIFTH - 1)
    plsc.subcore_barrier()

    wbase = s * ZROWS
    pltpu.sync_copy(acc_sh.at[pl.ds(wbase, ZROWS), :],
                    part_hbm.at[c, pl.ds(wbase, ZROWS), :])


def _combine_body(p_ref, o_ref):
    o_ref[...] = p_ref[0] + p_ref[1]


@jax.jit
def kernel(x, edge_index):
    n_pad = E_PAD - N_EDGES
    ar = jnp.arange(n_pad, dtype=jnp.int32)
    pad_src = ar % N_NODES
    pad_dst = N_NODES + (ar % (N_ACC - N_NODES))
    src2d = jnp.concatenate(
        [edge_index[0], pad_src]).reshape(N_ROWS, CHUNK)
    dst2d = jnp.concatenate(
        [edge_index[1], pad_dst]).reshape(N_ROWS, CHUNK)

    mesh = plsc.VectorSubcoreMesh(core_axis_name="c", subcore_axis_name="s",
                                  num_cores=NC, num_subcores=NS)
    partials = pl.kernel(
        _sc_accumulate,
        out_type=jax.ShapeDtypeStruct((NC, N_ACC, D_FEAT), jnp.float32),
        mesh=mesh,
        scratch_types=[
            pltpu.VMEM_SHARED((N_ACC, D_FEAT), jnp.float32),
            pltpu.VMEM((16, CHUNK), jnp.int32),
            pltpu.VMEM((16, CHUNK), jnp.int32),
            pltpu.VMEM((2, CHUNK, D_FEAT), jnp.float32),
            pltpu.SemaphoreType.DMA((2,)),
            pltpu.SemaphoreType.DMA((2,)),
        ],
    )(x, src2d, dst2d)

    out = pl.pallas_call(
        _combine_body,
        out_shape=jax.ShapeDtypeStruct((N_NODES, D_FEAT), jnp.float32),
        grid=(10,),
        in_specs=[pl.BlockSpec((NC, N_NODES // 10, D_FEAT),
                               lambda i: (0, i, 0))],
        out_specs=pl.BlockSpec((N_NODES // 10, D_FEAT), lambda i: (i, 0)),
    )(partials)
    return out
